# baseline (speedup 1.0000x reference)
import functools

import jax
import jax.numpy as jnp
from jax import lax
from jax.experimental import pallas as pl
from jax.experimental.pallas import tpu as pltpu
from jax.experimental.pallas import tpu_sc as plsc

LANES = 16


def _tc_build_packed(table_t, S4, C):
    D, V = table_t.shape
    G = S4 // C
    soff = S4 // C
    gmax = (V + C - 1) // C - 1

    def body(a_ref, b_ref, c_ref, d_ref, out_ref):
        x = jnp.concatenate(
            [a_ref[...], b_ref[...], c_ref[...], d_ref[...]], axis=0
        )
        x16 = x.astype(jnp.bfloat16)
        w = pltpu.bitcast(x16, jnp.float32)
        out_ref[...] = jnp.swapaxes(w, 0, 1)

    def mk_spec(q):
        return pl.BlockSpec(
            (D, C), lambda g: (0, jnp.minimum(g + q * soff, gmax))
        )

    return pl.pallas_call(
        body,
        grid=(G,),
        in_specs=[mk_spec(0), mk_spec(1), mk_spec(2), mk_spec(3)],
        out_specs=pl.BlockSpec((C, 2 * D), lambda g: (g, 0)),
        out_shape=jax.ShapeDtypeStruct((S4, 2 * D), jnp.float32),
    )(table_t, table_t, table_t, table_t)


def _sc_gather_classify(ids, table_lin, wt, bpad, S4, D, L_true, OUT):
    L, B = ids.shape
    NC, NS = 2, 16
    NW = NC * NS
    bpw = B // NW

    mesh = plsc.VectorSubcoreMesh(core_axis_name="c", subcore_axis_name="s")

    @functools.partial(
        pl.kernel,
        out_type=jax.ShapeDtypeStruct((B, OUT), jnp.float32),
        mesh=mesh,
        scratch_types=[
            pltpu.VMEM((L, bpw), jnp.int32),
            pltpu.VMEM((bpw * ((L + 7) // 8 * 8),), jnp.int32),
            pltpu.VMEM((4, L, D // 2), jnp.float32),
            pltpu.VMEM((bpw, D), jnp.float32),
            pltpu.VMEM((OUT, D), jnp.float32),
            pltpu.VMEM((16,), jnp.float32),
            pltpu.VMEM((bpw, OUT), jnp.float32),
            pltpu.SemaphoreType.DMA,
            pltpu.SemaphoreType.DMA,
            pltpu.SemaphoreType.DMA,
            pltpu.SemaphoreType.DMA,
        ],
        compiler_params=pltpu.CompilerParams(
            use_tc_tiling_on_sc=False, needs_layout_passes=False
        ),
    )
    def k(
        ids_hbm, table_hbm, wt_hbm, b_hbm, out_hbm, idx_v, idxt_v, rows_v,
        acc_v, wt_v, bv_v, log_v, sem0, sem1, sem2, sem3,
    ):
        wid = lax.axis_index("s") * NC + lax.axis_index("c")
        base = wid * bpw
        pltpu.sync_copy(ids_hbm.at[:, pl.ds(base, bpw)], idx_v)
        pltpu.sync_copy(wt_hbm, wt_v)
        pltpu.sync_copy(b_hbm, bv_v)
        sems = (sem0, sem1, sem2, sem3)

        lanes_i = lax.iota(jnp.int32, LANES)
        LP = (L + 7) // 8 * 8

        def hbody(l, _):
            for c in range(bpw // LANES):
                sl = pl.ds(c * LANES, LANES)
                v = idx_v[l, sl]
                one = jnp.ones((LANES,), jnp.int32)
                zero = jnp.zeros((LANES,), jnp.int32)
                q = jnp.where(v >= S4, one, zero)
                q = q + jnp.where(v >= 2 * S4, one, zero)
                q = q + jnp.where(v >= 3 * S4, one, zero)
                r = v - q * S4
                offs = (lanes_i + c * LANES) * LP + l
                plsc.store_scatter(idxt_v, [offs], 4 * r + q)
            return 0

        lax.fori_loop(0, L, hbody, 0)

        def gather(j, buf):
            pltpu.async_copy(
                table_hbm.at[idxt_v.at[pl.ds(j * LP, L)]], rows_v.at[buf], sems[buf]
            )

        def wait_gather(j, buf):
            pltpu.make_async_copy(
                table_hbm.at[idxt_v.at[pl.ds(j * LP, L)]], rows_v.at[buf], sems[buf]
            ).wait()

        def consume(j, buf):
            accs = None
            for l in range(L):
                w0 = rows_v[buf, l, pl.ds(0, LANES)]
                w1 = rows_v[buf, l, pl.ds(LANES, LANES)]
                l0, h0 = plsc.unpack(
                    plsc.bitcast(w0, jnp.bfloat16),
                    format=plsc.PackFormat.INTERLEAVED,
                )
                l1, h1 = plsc.unpack(
                    plsc.bitcast(w1, jnp.bfloat16),
                    format=plsc.PackFormat.INTERLEAVED,
                )
                parts = (l0, h0, l1, h1)
                if accs is None:
                    accs = parts
                else:
                    accs = tuple(a + p for a, p in zip(accs, parts))
            for c, a in enumerate(accs):
                acc_v[j, pl.ds(c * LANES, LANES)] = a

        NB = 4
        for b in range(NB):
            gather(b, b)

        def jmain(g, _):
            for u in range(NB):
                j = NB * g + u
                wait_gather(j, u)
                consume(j, u)

                @pl.when(j + NB < bpw)
                def _():
                    gather(j + NB, u)

            return 0

        lax.fori_loop(0, bpw // NB, jmain, 0)

        bvec = bv_v[...]
        inv_l = 1.0 / L_true

        def cbody(g, _):
            jvec = g * LANES + lanes_i
            zs = [jnp.zeros((LANES,), jnp.float32) for _ in range(OUT)]
            wvecs = [
                [wt_v[o, pl.ds(c * LANES, LANES)] for c in range(D // LANES)]
                for o in range(OUT)
            ]
            for p in range(D):
                pvec = jnp.zeros((LANES,), jnp.int32) + p
                col = plsc.load_gather(acc_v, [jvec, pvec])
                for o in range(OUT):
                    zs[o] = zs[o] + col * wvecs[o][p // LANES][p % LANES]
            for o in range(OUT):
                z = zs[o] * inv_l + bvec[o]
                s = 1.0 / (1.0 + jnp.exp(-z))
                ovec = jnp.zeros((LANES,), jnp.int32) + o
                plsc.store_scatter(log_v, [jvec, ovec], s)
            return 0

        lax.fori_loop(0, bpw // LANES, cbody, 0)
        pltpu.sync_copy(log_v, out_hbm.at[pl.ds(base, bpw)])

    return k(ids, table_lin, wt, bpad)


def _tc_classifier(sums, cls_w, cls_b, L):
    B, D = sums.shape
    _, OUT = cls_w.shape

    def body(x_ref, w_ref, b_ref, o_ref):
        x = x_ref[...] * (1.0 / L)
        y = jnp.dot(x, w_ref[...], preferred_element_type=jnp.float32)
        o_ref[...] = jax.nn.sigmoid(y + b_ref[...])

    return pl.pallas_call(
        body,
        out_shape=jax.ShapeDtypeStruct((B, OUT), jnp.float32),
    )(sums, cls_w, cls_b.reshape(1, OUT))


def kernel(input_ids, emb_table, cls_w, cls_b):
    ids = input_ids.astype(jnp.int32)
    L, _ = ids.shape
    V, D = emb_table.shape
    C = 8192
    S4 = C * (((V + 3) // 4 + C - 1) // C)
    tablev = _tc_build_packed(emb_table.T, S4, C)
    table_lin = tablev.reshape(4 * S4, D // 2)
    perm = []
    for half in range(D // 32):
        perm += [32 * half + 2 * k for k in range(16)]
        perm += [32 * half + 2 * k + 1 for k in range(16)]
    wt = cls_w[jnp.array(perm, dtype=jnp.int32), :].T
    bpad = jnp.pad(cls_b, (0, 16 - cls_b.shape[0])).astype(jnp.float32)
    OUT = cls_w.shape[1]
    return _sc_gather_classify(ids, table_lin, wt, bpad, S4, D, L, OUT)


# speedup vs baseline: 1.0222x; 1.0222x over previous
import functools

import jax
import jax.numpy as jnp
from jax import lax
from jax.experimental import pallas as pl
from jax.experimental.pallas import tpu as pltpu
from jax.experimental.pallas import tpu_sc as plsc

LANES = 16


def _tc_build_packed(table_t, S4, C):
    D, V = table_t.shape
    G = S4 // C
    soff = S4 // C
    gmax = (V + C - 1) // C - 1

    def body(a_ref, b_ref, c_ref, d_ref, out_ref):
        x = jnp.concatenate(
            [a_ref[...], b_ref[...], c_ref[...], d_ref[...]], axis=0
        )
        x16 = x.astype(jnp.bfloat16)
        w = pltpu.bitcast(x16, jnp.float32)
        out_ref[...] = jnp.swapaxes(w, 0, 1)

    def mk_spec(q):
        return pl.BlockSpec(
            (D, C), lambda g: (0, jnp.minimum(g + q * soff, gmax))
        )

    return pl.pallas_call(
        body,
        grid=(G,),
        in_specs=[mk_spec(0), mk_spec(1), mk_spec(2), mk_spec(3)],
        out_specs=pl.BlockSpec((C, 2 * D), lambda g: (g, 0)),
        out_shape=jax.ShapeDtypeStruct((S4, 2 * D), jnp.float32),
    )(table_t, table_t, table_t, table_t)


def _sc_gather_sum(ids, table_lin, S4, D):
    L, B = ids.shape
    W = D // 4
    NC, NS = 2, 16
    NW = NC * NS
    bpw = B // NW
    JU = 16

    mesh = plsc.VectorSubcoreMesh(core_axis_name="c", subcore_axis_name="s")

    @functools.partial(
        pl.kernel,
        out_type=jax.ShapeDtypeStruct((B, D), jnp.float32),
        mesh=mesh,
        scratch_types=[
            pltpu.VMEM((L, bpw), jnp.int32),
            pltpu.VMEM((bpw * ((L + 7) // 8 * 8),), jnp.int32),
            pltpu.VMEM((4, L, D // 2), jnp.float32),
            pltpu.VMEM((bpw, D), jnp.float32),
            pltpu.SemaphoreType.DMA,
            pltpu.SemaphoreType.DMA,
            pltpu.SemaphoreType.DMA,
            pltpu.SemaphoreType.DMA,
        ],
        compiler_params=pltpu.CompilerParams(
            use_tc_tiling_on_sc=False, needs_layout_passes=False
        ),
    )
    def k(
        ids_hbm, table_hbm, out_hbm, idx_v, idxt_v, rows_v, acc_v,
        sem0, sem1, sem2, sem3,
    ):
        wid = lax.axis_index("s") * NC + lax.axis_index("c")
        base = wid * bpw
        pltpu.sync_copy(ids_hbm.at[:, pl.ds(base, bpw)], idx_v)
        sems = (sem0, sem1, sem2, sem3)

        lanes_i = lax.iota(jnp.int32, LANES)
        LP = (L + 7) // 8 * 8

        def hbody(l, _):
            for c in range(bpw // LANES):
                sl = pl.ds(c * LANES, LANES)
                v = idx_v[l, sl]
                one = jnp.ones((LANES,), jnp.int32)
                zero = jnp.zeros((LANES,), jnp.int32)
                q = jnp.where(v >= S4, one, zero)
                q = q + jnp.where(v >= 2 * S4, one, zero)
                q = q + jnp.where(v >= 3 * S4, one, zero)
                r = v - q * S4
                offs = (lanes_i + c * LANES) * LP + l
                plsc.store_scatter(idxt_v, [offs], 4 * r + q)
            return 0

        lax.fori_loop(0, L, hbody, 0)

        def gather(j, buf):
            pltpu.async_copy(
                table_hbm.at[idxt_v.at[pl.ds(j * LP, L)]], rows_v.at[buf], sems[buf]
            )

        def wait_gather(j, buf):
            pltpu.make_async_copy(
                table_hbm.at[idxt_v.at[pl.ds(j * LP, L)]], rows_v.at[buf], sems[buf]
            ).wait()

        def consume(j, buf):
            accs = None
            for l in range(L):
                w0 = rows_v[buf, l, pl.ds(0, LANES)]
                w1 = rows_v[buf, l, pl.ds(LANES, LANES)]
                l0, h0 = plsc.unpack(
                    plsc.bitcast(w0, jnp.bfloat16),
                    format=plsc.PackFormat.INTERLEAVED,
                )
                l1, h1 = plsc.unpack(
                    plsc.bitcast(w1, jnp.bfloat16),
                    format=plsc.PackFormat.INTERLEAVED,
                )
                parts = (l0, h0, l1, h1)
                if accs is None:
                    accs = parts
                else:
                    accs = tuple(a + p for a, p in zip(accs, parts))
            for c, a in enumerate(accs):
                acc_v[j, pl.ds(c * LANES, LANES)] = a

        NB = 4
        for b in range(NB):
            gather(b, b)

        def jmain(g, _):
            for u in range(NB):
                j = NB * g + u
                wait_gather(j, u)
                consume(j, u)

                @pl.when(j + NB < bpw)
                def _():
                    gather(j + NB, u)

            return 0

        lax.fori_loop(0, bpw // NB, jmain, 0)
        pltpu.sync_copy(acc_v, out_hbm.at[pl.ds(base, bpw)])

    return k(ids, table_lin)


def _tc_classifier(sums, cls_w, cls_b, L):
    B, D = sums.shape
    _, OUT = cls_w.shape

    def body(x_ref, w_ref, b_ref, o_ref):
        x = x_ref[...] * (1.0 / L)
        y = jnp.dot(x, w_ref[...], preferred_element_type=jnp.float32)
        o_ref[...] = jax.nn.sigmoid(y + b_ref[...])

    return pl.pallas_call(
        body,
        out_shape=jax.ShapeDtypeStruct((B, OUT), jnp.float32),
    )(sums, cls_w, cls_b.reshape(1, OUT))


def kernel(input_ids, emb_table, cls_w, cls_b):
    ids = input_ids.astype(jnp.int32)
    L, _ = ids.shape
    V, D = emb_table.shape
    C = 8192
    S4 = C * (((V + 3) // 4 + C - 1) // C)
    tablev = _tc_build_packed(emb_table.T, S4, C)
    table_lin = tablev.reshape(4 * S4, D // 2)
    sums = _sc_gather_sum(ids, table_lin, S4, D)
    perm = []
    for half in range(D // 32):
        perm += [32 * half + 2 * k for k in range(16)]
        perm += [32 * half + 2 * k + 1 for k in range(16)]
    cls_w_p = cls_w[jnp.array(perm, dtype=jnp.int32), :]
    return _tc_classifier(sums, cls_w_p, cls_b, L)
